# baseline (speedup 1.0000x reference)
import functools

import jax
import jax.numpy as jnp
from jax import lax
from jax.experimental import pallas as pl
from jax.experimental.pallas import tpu as pltpu
from jax.experimental.pallas import tpu_sc as plsc

_B = 128
_N = 32768
_L = 16
_NSLICES = _N // _L
_NWORKERS = 32
_ROWS_PER_WORKER = _B // _NWORKERS
_CAP = 8192

_mesh = plsc.VectorSubcoreMesh(core_axis_name="c", subcore_axis_name="s")


@functools.partial(
    pl.kernel,
    out_type=jax.ShapeDtypeStruct((_B, _N), jnp.float32),
    mesh=_mesh,
    scratch_types=[
        pltpu.VMEM((_N,), jnp.float32),
        pltpu.VMEM((_CAP + _L,), jnp.int32),
    ],
    compiler_params=pltpu.CompilerParams(needs_layout_passes=False),
)
def _sparsemax_sc(x_hbm, out_hbm, row_v, cidx_v):
    wid = lax.axis_index("s") * 2 + lax.axis_index("c")
    iota = lax.broadcasted_iota(jnp.int32, (_L,), 0)
    zf = jnp.zeros((_L,), jnp.float32)

    for r in range(_ROWS_PER_WORKER):
        row = wid * _ROWS_PER_WORKER + r
        pltpu.sync_copy(x_hbm.at[row], row_v)

        def fuse_body(i, carry):
            vmax_m1, off = carry
            v = row_v[pl.ds(i * _L, _L)]
            msk = v > vmax_m1
            vmax_m1 = jnp.maximum(vmax_m1, v - 1.0)
            cnt = plsc.all_reduce_population_count(msk)
            pos = off + plsc.cumsum(jnp.where(msk, 1, 0)) - 1
            pos = jnp.minimum(pos, _CAP - 1)
            plsc.store_scatter(cidx_v, [pos], i * _L + iota, mask=msk)
            return (vmax_m1, off + cnt)

        vmax_m1, off = lax.fori_loop(
            0, _NSLICES, fuse_body,
            (jnp.full((_L,), -3.0e38, jnp.float32), jnp.zeros((_L,), jnp.int32)),
            unroll=8)
        n = off[0]
        m = jnp.max(vmax_m1) + 1.0
        t0 = m - 1.0

        def michelot(fk):
            def cond(carry):
                t_prev, t = carry
                return t > t_prev

            def body(carry):
                _, t = carry
                s, k = fk(t)
                t_new = ((jnp.full((_L,), s) - 1.0) / jnp.full((_L,), k))[0]
                return (t, jnp.maximum(t, t_new))

            return lax.while_loop(cond, body, (t0 - 1.0, t0))[1]

        def fast_tau(_):
            ncs = (n + _L - 1) // _L

            def fk(t):
                def b(j, acc):
                    s_acc, k_acc = acc
                    tl = (j * _L + iota) < n
                    idx = jnp.where(tl, cidx_v[pl.ds(j * _L, _L)], 0)
                    v = plsc.load_gather(row_v, [idx])
                    msk = (v > t) & tl
                    return (s_acc + jnp.where(msk, v, 0.0),
                            k_acc + jnp.where(msk, 1.0, 0.0))

                s_acc, k_acc = lax.fori_loop(0, ncs, b, (zf, zf))
                return jnp.sum(s_acc), jnp.sum(k_acc)

            return michelot(fk)

        def slow_tau(_):
            def fk(t):
                def b(j, acc):
                    s_acc, k_acc = acc
                    v = row_v[pl.ds(j * _L, _L)]
                    msk = v > t
                    return (s_acc + jnp.where(msk, v, 0.0),
                            k_acc + jnp.where(msk, 1.0, 0.0))

                s_acc, k_acc = lax.fori_loop(0, _NSLICES, b, (zf, zf))
                return jnp.sum(s_acc), jnp.sum(k_acc)

            return michelot(fk)

        tau = lax.cond(n <= _CAP, fast_tau, slow_tau, 0)

        def out_body(i, carry):
            v = row_v[pl.ds(i * _L, _L)]
            row_v[pl.ds(i * _L, _L)] = jnp.maximum(v - tau, 0.0)
            return carry

        lax.fori_loop(0, _NSLICES, out_body, 0, unroll=8)
        pltpu.sync_copy(row_v, out_hbm.at[row])


def kernel(input):
    return _sparsemax_sc(input)


# speedup vs baseline: 1.4116x; 1.4116x over previous
import functools

import jax
import jax.numpy as jnp
from jax import lax
from jax.experimental import pallas as pl
from jax.experimental.pallas import tpu as pltpu
from jax.experimental.pallas import tpu_sc as plsc

_B = 128
_N = 32768
_L = 16
_NSLICES = _N // _L
_NWORKERS = 32
_ROWS_PER_WORKER = _B // _NWORKERS
_CAPL = 1024

_mesh = plsc.VectorSubcoreMesh(core_axis_name="c", subcore_axis_name="s")


@functools.partial(
    pl.kernel,
    out_type=jax.ShapeDtypeStruct((_B, _N), jnp.float32),
    mesh=_mesh,
    scratch_types=[
        pltpu.VMEM((_N,), jnp.float32),
        pltpu.VMEM((_CAPL * _L,), jnp.int32),
    ],
    compiler_params=pltpu.CompilerParams(needs_layout_passes=False),
)
def _sparsemax_sc(x_hbm, out_hbm, row_v, cidx_v):
    wid = lax.axis_index("s") * 2 + lax.axis_index("c")
    iota = lax.broadcasted_iota(jnp.int32, (_L,), 0)
    zf = jnp.zeros((_L,), jnp.float32)
    pos_lim = jnp.int32((_CAPL - 1) * _L)

    for r in range(_ROWS_PER_WORKER):
        row = wid * _ROWS_PER_WORKER + r
        pltpu.sync_copy(x_hbm.at[row], row_v)

        def fuse_body(i, carry):
            vmax_m1, pos = carry
            v = row_v[pl.ds(i * _L, _L)]
            msk = v > vmax_m1
            vmax_m1 = jnp.maximum(vmax_m1, v - 1.0)
            plsc.store_scatter(cidx_v, [jnp.minimum(pos, pos_lim + iota)],
                               i * _L + iota, mask=msk)
            pos = pos + jnp.where(msk, _L, 0)
            return (vmax_m1, pos)

        vmax_m1, pos = lax.fori_loop(
            0, _NSLICES, fuse_body,
            (jnp.full((_L,), -3.0e38, jnp.float32), iota),
            unroll=8)
        lane_cnt = (pos - iota) // _L
        maxcnt = jnp.max(lane_cnt)
        m = jnp.max(vmax_m1) + 1.0
        t0 = m - 1.0

        def michelot(fk):
            def cond(carry):
                t_prev, t = carry
                return t > t_prev

            def body(carry):
                _, t = carry
                s, k = fk(t)
                t_new = ((jnp.full((_L,), s) - 1.0) / jnp.full((_L,), k))[0]
                return (t, jnp.maximum(t, t_new))

            return lax.while_loop(cond, body, (t0 - 1.0, t0))[1]

        def fast_tau(_):
            def fk(t):
                def b(j, acc):
                    s_acc, k_acc = acc
                    valid = j < lane_cnt
                    idx = jnp.where(valid, cidx_v[pl.ds(j * _L, _L)], 0)
                    v = plsc.load_gather(row_v, [idx])
                    msk = (v > t) & valid
                    return (s_acc + jnp.where(msk, v, 0.0),
                            k_acc + jnp.where(msk, 1.0, 0.0))

                s_acc, k_acc = lax.fori_loop(0, maxcnt, b, (zf, zf))
                return jnp.sum(s_acc), jnp.sum(k_acc)

            return michelot(fk)

        def slow_tau(_):
            def fk(t):
                def b(j, acc):
                    s_acc, k_acc = acc
                    v = row_v[pl.ds(j * _L, _L)]
                    msk = v > t
                    return (s_acc + jnp.where(msk, v, 0.0),
                            k_acc + jnp.where(msk, 1.0, 0.0))

                s_acc, k_acc = lax.fori_loop(0, _NSLICES, b, (zf, zf))
                return jnp.sum(s_acc), jnp.sum(k_acc)

            return michelot(fk)

        tau = lax.cond(maxcnt <= _CAPL, fast_tau, slow_tau, 0)

        def out_body(i, carry):
            v = row_v[pl.ds(i * _L, _L)]
            row_v[pl.ds(i * _L, _L)] = jnp.maximum(v - tau, 0.0)
            return carry

        lax.fori_loop(0, _NSLICES, out_body, 0, unroll=8)
        pltpu.sync_copy(row_v, out_hbm.at[row])


def kernel(input):
    return _sparsemax_sc(input)


# speedup vs baseline: 2.5358x; 1.7964x over previous
import functools

import jax
import jax.numpy as jnp
from jax import lax
from jax.experimental import pallas as pl
from jax.experimental.pallas import tpu as pltpu
from jax.experimental.pallas import tpu_sc as plsc

_B = 128
_N = 32768
_L = 16
_NSLICES = _N // _L
_NWORKERS = 32
_ROWS_PER_WORKER = _B // _NWORKERS
_CAPL = 1024

_mesh = plsc.VectorSubcoreMesh(core_axis_name="c", subcore_axis_name="s")


@functools.partial(
    pl.kernel,
    out_type=jax.ShapeDtypeStruct((_B, _N), jnp.float32),
    mesh=_mesh,
    scratch_types=[
        pltpu.VMEM((_N,), jnp.float32),
        pltpu.VMEM((_CAPL * _L,), jnp.float32),
    ],
    compiler_params=pltpu.CompilerParams(needs_layout_passes=False),
)
def _sparsemax_sc(x_hbm, out_hbm, row_v, cval_v):
    wid = lax.axis_index("s") * 2 + lax.axis_index("c")
    iota = lax.broadcasted_iota(jnp.int32, (_L,), 0)
    zf = jnp.zeros((_L,), jnp.float32)
    wrap = jnp.int32(_CAPL * _L - 1)

    for r in range(_ROWS_PER_WORKER):
        row = wid * _ROWS_PER_WORKER + r
        pltpu.sync_copy(x_hbm.at[row], row_v)

        def _fuse(i, carry):
            vmax_m1, pos = carry
            v = row_v[pl.ds(i * _L, _L)]
            msk = v > vmax_m1
            vmax_m1 = jnp.maximum(vmax_m1, v - 1.0)
            plsc.store_scatter(cval_v, [pos & wrap], v, mask=msk)
            pos = pos + jnp.where(msk, _L, 0)
            return (vmax_m1, pos)

        vmax_m1, pos = plsc.parallel_loop(
            0, _NSLICES, unroll=8,
            carry=(jnp.full((_L,), -3.0e38, jnp.float32), iota))(_fuse)
        lane_cnt = (pos - iota) // _L
        maxcnt = jnp.max(lane_cnt)
        m = jnp.max(vmax_m1) + 1.0
        t0 = m - 1.0

        def michelot(fk):
            def cond(carry):
                t_prev, t = carry
                return t > t_prev

            def body(carry):
                _, t = carry
                s, k = fk(t)
                t_new = ((jnp.full((_L,), s) - 1.0) / jnp.full((_L,), k))[0]
                return (t, jnp.maximum(t, t_new))

            return lax.while_loop(cond, body, (t0 - 1.0, t0))[1]

        def fast_tau(_):
            def fk(t):
                def b(j, acc):
                    s_acc, k_acc = acc
                    v = cval_v[pl.ds(j * _L, _L)]
                    msk = (v > t) & (j < lane_cnt)
                    return (s_acc + jnp.where(msk, v, 0.0),
                            k_acc + jnp.where(msk, 1.0, 0.0))

                s_acc, k_acc = lax.fori_loop(0, maxcnt, b, (zf, zf))
                return jnp.sum(s_acc), jnp.sum(k_acc)

            return michelot(fk)

        def slow_tau(_):
            def fk(t):
                def b(j, acc):
                    s_acc, k_acc = acc
                    v = row_v[pl.ds(j * _L, _L)]
                    msk = v > t
                    return (s_acc + jnp.where(msk, v, 0.0),
                            k_acc + jnp.where(msk, 1.0, 0.0))

                s_acc, k_acc = lax.fori_loop(0, _NSLICES, b, (zf, zf))
                return jnp.sum(s_acc), jnp.sum(k_acc)

            return michelot(fk)

        tau = lax.cond(maxcnt <= _CAPL, fast_tau, slow_tau, 0)

        def _out(i):
            v = row_v[pl.ds(i * _L, _L)]
            row_v[pl.ds(i * _L, _L)] = jnp.maximum(v - tau, 0.0)

        plsc.parallel_loop(0, _NSLICES, unroll=8)(_out)

        pltpu.sync_copy(row_v, out_hbm.at[row])


def kernel(input):
    return _sparsemax_sc(input)


# speedup vs baseline: 2.8777x; 1.1348x over previous
import functools

import jax
import jax.numpy as jnp
from jax import lax
from jax.experimental import pallas as pl
from jax.experimental.pallas import tpu as pltpu
from jax.experimental.pallas import tpu_sc as plsc

_B = 128
_N = 32768
_L = 16
_NSLICES = _N // _L
_NWORKERS = 32
_ROWS_PER_WORKER = _B // _NWORKERS
_CAPL = 1024

_mesh = plsc.VectorSubcoreMesh(core_axis_name="c", subcore_axis_name="s")


@functools.partial(
    pl.kernel,
    out_type=jax.ShapeDtypeStruct((_B, _N), jnp.float32),
    mesh=_mesh,
    scratch_types=[
        pltpu.VMEM((_N,), jnp.float32),
        pltpu.VMEM((_N,), jnp.float32),
        pltpu.VMEM((_N,), jnp.float32),
        pltpu.VMEM((_CAPL * _L,), jnp.float32),
        pltpu.SemaphoreType.DMA,
        pltpu.SemaphoreType.DMA,
        pltpu.SemaphoreType.DMA,
        pltpu.SemaphoreType.DMA,
        pltpu.SemaphoreType.DMA,
        pltpu.SemaphoreType.DMA,
    ],
    compiler_params=pltpu.CompilerParams(needs_layout_passes=False),
)
def _sparsemax_sc(x_hbm, out_hbm, row0_v, row1_v, row2_v, cval_v,
                  isem0, isem1, isem2, osem0, osem1, osem2):
    wid = lax.axis_index("s") * 2 + lax.axis_index("c")
    iota = lax.broadcasted_iota(jnp.int32, (_L,), 0)
    zf = jnp.zeros((_L,), jnp.float32)
    wrap = jnp.int32(_CAPL * _L - 1)

    bufs = [row0_v, row1_v, row2_v]
    isems = [isem0, isem1, isem2]
    osems = [osem0, osem1, osem2]
    in_h = {r: pltpu.async_copy(x_hbm.at[wid * _ROWS_PER_WORKER + r],
                                bufs[r], isems[r])
            for r in range(3)}
    out_h = {}

    for r in range(_ROWS_PER_WORKER):
        row = wid * _ROWS_PER_WORKER + r
        row_v = bufs[r % 3]
        in_h[r].wait()

        def _fuse(i, carry):
            vmax_m1, pos = carry
            v = row_v[pl.ds(i * _L, _L)]
            msk = v > vmax_m1
            vmax_m1 = jnp.maximum(vmax_m1, v - 1.0)
            plsc.store_scatter(cval_v, [pos & wrap], v, mask=msk)
            pos = pos + jnp.where(msk, _L, 0)
            return (vmax_m1, pos)

        vmax_m1, pos = plsc.parallel_loop(
            0, _NSLICES, unroll=8,
            carry=(jnp.full((_L,), -3.0e38, jnp.float32), iota))(_fuse)

        nxt = r + 2
        if 3 <= nxt < _ROWS_PER_WORKER:
            out_h[nxt - 3].wait()
            in_h[nxt] = pltpu.async_copy(
                x_hbm.at[wid * _ROWS_PER_WORKER + nxt], bufs[nxt % 3],
                isems[nxt % 3])
        lane_cnt = (pos - iota) // _L
        maxcnt = jnp.max(lane_cnt)
        m = jnp.max(vmax_m1) + 1.0
        t0 = m - 1.0

        def michelot(fk):
            def cond(carry):
                t_prev, t = carry
                return t > t_prev

            def body(carry):
                _, t = carry
                s, k = fk(t)
                t_new = ((jnp.full((_L,), s) - 1.0) / jnp.full((_L,), k))[0]
                return (t, jnp.maximum(t, t_new))

            return lax.while_loop(cond, body, (t0 - 1.0, t0))[1]

        def fast_tau(_):
            def fk(t):
                def b(j, acc):
                    s_acc, k_acc = acc
                    v = cval_v[pl.ds(j * _L, _L)]
                    msk = (v > t) & (j < lane_cnt)
                    return (s_acc + jnp.where(msk, v, 0.0),
                            k_acc + jnp.where(msk, 1.0, 0.0))

                s_acc, k_acc = lax.fori_loop(0, maxcnt, b, (zf, zf))
                return jnp.sum(s_acc), jnp.sum(k_acc)

            return michelot(fk)

        def slow_tau(_):
            def fk(t):
                def b(j, acc):
                    s_acc, k_acc = acc
                    v = row_v[pl.ds(j * _L, _L)]
                    msk = v > t
                    return (s_acc + jnp.where(msk, v, 0.0),
                            k_acc + jnp.where(msk, 1.0, 0.0))

                s_acc, k_acc = lax.fori_loop(0, _NSLICES, b, (zf, zf))
                return jnp.sum(s_acc), jnp.sum(k_acc)

            return michelot(fk)

        tau = lax.cond(maxcnt <= _CAPL, fast_tau, slow_tau, 0)

        def _out(i):
            v = row_v[pl.ds(i * _L, _L)]
            row_v[pl.ds(i * _L, _L)] = jnp.maximum(v - tau, 0.0)

        plsc.parallel_loop(0, _NSLICES, unroll=8)(_out)

        out_h[r] = pltpu.async_copy(row_v, out_hbm.at[row], osems[r % 3])

    for r in range(max(0, _ROWS_PER_WORKER - 3), _ROWS_PER_WORKER):
        out_h[r].wait()


def kernel(input):
    return _sparsemax_sc(input)
